# baseline (speedup 1.0000x reference)
import jax
import jax.numpy as jnp
from jax.experimental import pallas as pl
from jax.experimental.pallas import tpu as pltpu

_N = 8192
_K = 4096
_B = 128
_R = 64
_C = 128
_ROWS = 32


def _vsum(ind):
    s = jnp.sum(jnp.where(ind, jnp.int32(1), jnp.int32(0)), axis=0,
                keepdims=True)
    return jnp.sum(s, axis=1, keepdims=True)


def _body(w_ref, x_ref, mask_ref, o_ref, mvec_ref):
    step = pl.program_id(0)

    @pl.when(step == 0)
    def _select():
        v = jnp.abs(w_ref[...])
        u = jax.lax.bitcast_convert_type(v, jnp.int32)

        t = jnp.zeros((1, 1), jnp.int32)
        for b, hi in ((28, 7), (24, 15), (20, 15), (16, 15),
                      (12, 15), (8, 15), (4, 15), (0, 15)):
            d = jnp.zeros((1, 1), jnp.int32)
            for j in range(1, hi + 1):
                cnt = _vsum(u >= (t | jnp.int32(j << b)))
                d = d + jnp.where(cnt >= _K, jnp.int32(1), jnp.int32(0))
            t = t | jax.lax.shift_left(d, b)

        gt = u > t
        eq = u == t
        ties = (_K - _vsum(gt)).astype(jnp.float32)

        eqf = jnp.where(eq, jnp.float32(1.0), jnp.float32(0.0))
        jj = jax.lax.broadcasted_iota(jnp.int32, (_C, _C), 0)
        cc = jax.lax.broadcasted_iota(jnp.int32, (_C, _C), 1)
        tri_c = jnp.where(jj < cc, jnp.float32(1.0), jnp.float32(0.0))
        inrow = jnp.dot(eqf, tri_c, preferred_element_type=jnp.float32)
        rowsum = jnp.sum(eqf, axis=1, keepdims=True)
        r0 = jax.lax.broadcasted_iota(jnp.int32, (_R, _R), 0)
        r1 = jax.lax.broadcasted_iota(jnp.int32, (_R, _R), 1)
        tri_r = jnp.where(r1 < r0, jnp.float32(1.0), jnp.float32(0.0))
        rowpre = jnp.dot(tri_r, rowsum, preferred_element_type=jnp.float32)
        prefix = inrow + rowpre

        keep = gt | (eq & (prefix < ties))
        maskf = jnp.where(keep, jnp.float32(1.0), jnp.float32(0.0))
        for r in range(_R):
            mvec_ref[0:1, r * _C:(r + 1) * _C] = maskf[r:r + 1, :]
        mask_ref[...] = mvec_ref[...]

    o_ref[...] = x_ref[...] * mvec_ref[...]


def kernel(x, weights):
    w2 = weights.reshape(_R, _C)
    mask2, sel = pl.pallas_call(
        _body,
        grid=(_B // _ROWS,),
        in_specs=[
            pl.BlockSpec((_R, _C), lambda i: (0, 0)),
            pl.BlockSpec((_ROWS, _N), lambda i: (i, 0)),
        ],
        out_specs=[
            pl.BlockSpec((1, _N), lambda i: (0, 0)),
            pl.BlockSpec((_ROWS, _N), lambda i: (i, 0)),
        ],
        out_shape=[
            jax.ShapeDtypeStruct((1, _N), jnp.float32),
            jax.ShapeDtypeStruct((_B, _N), jnp.float32),
        ],
        scratch_shapes=[pltpu.VMEM((1, _N), jnp.float32)],
    )(w2, x)
    return (sel, mask2.reshape(_N))


# speedup vs baseline: 1.1952x; 1.1952x over previous
import jax
import jax.numpy as jnp
from jax.experimental import pallas as pl
from jax.experimental.pallas import tpu as pltpu

_N = 8192
_K = 4096
_B = 128
_R = 64
_C = 128
_ROWS = 64


def _vsum(ind):
    s = jnp.sum(jnp.where(ind, jnp.int32(1), jnp.int32(0)), axis=0,
                keepdims=True)
    return jnp.sum(s, axis=1, keepdims=True)


def _body(w_ref, x_ref, mask_ref, o_ref, mvec_ref):
    step = pl.program_id(0)

    @pl.when(step == 0)
    def _select():
        v = jnp.abs(w_ref[...])
        u = jax.lax.bitcast_convert_type(v, jnp.int32)

        t = jnp.zeros((1, 1), jnp.int32)
        for b, hi in ((28, 7), (24, 15), (20, 15), (16, 15),
                      (12, 15), (8, 15), (4, 15), (0, 15)):
            d = jnp.zeros((1, 1), jnp.int32)
            for j in range(1, hi + 1):
                cnt = _vsum(u >= (t | jnp.int32(j << b)))
                d = d + jnp.where(cnt >= _K, jnp.int32(1), jnp.int32(0))
            t = t | jax.lax.shift_left(d, b)

        gt = u > t
        eq = u == t
        ties = (_K - _vsum(gt)).astype(jnp.float32)

        eqf = jnp.where(eq, jnp.float32(1.0), jnp.float32(0.0))
        jj = jax.lax.broadcasted_iota(jnp.int32, (_C, _C), 0)
        cc = jax.lax.broadcasted_iota(jnp.int32, (_C, _C), 1)
        tri_c = jnp.where(jj < cc, jnp.float32(1.0), jnp.float32(0.0))
        inrow = jnp.dot(eqf, tri_c, preferred_element_type=jnp.float32)
        rowsum = jnp.sum(eqf, axis=1, keepdims=True)
        r0 = jax.lax.broadcasted_iota(jnp.int32, (_R, _R), 0)
        r1 = jax.lax.broadcasted_iota(jnp.int32, (_R, _R), 1)
        tri_r = jnp.where(r1 < r0, jnp.float32(1.0), jnp.float32(0.0))
        rowpre = jnp.dot(tri_r, rowsum, preferred_element_type=jnp.float32)
        prefix = inrow + rowpre

        keep = gt | (eq & (prefix < ties))
        maskf = jnp.where(keep, jnp.float32(1.0), jnp.float32(0.0))
        for r in range(_R):
            mvec_ref[0:1, r * _C:(r + 1) * _C] = maskf[r:r + 1, :]
        mask_ref[...] = mvec_ref[...]

    o_ref[...] = x_ref[...] * mvec_ref[...]


def kernel(x, weights):
    w2 = weights.reshape(_R, _C)
    mask2, sel = pl.pallas_call(
        _body,
        grid=(_B // _ROWS,),
        in_specs=[
            pl.BlockSpec((_R, _C), lambda i: (0, 0)),
            pl.BlockSpec((_ROWS, _N), lambda i: (i, 0)),
        ],
        out_specs=[
            pl.BlockSpec((1, _N), lambda i: (0, 0)),
            pl.BlockSpec((_ROWS, _N), lambda i: (i, 0)),
        ],
        out_shape=[
            jax.ShapeDtypeStruct((1, _N), jnp.float32),
            jax.ShapeDtypeStruct((_B, _N), jnp.float32),
        ],
        scratch_shapes=[pltpu.VMEM((1, _N), jnp.float32)],
    )(w2, x)
    return (sel, mask2.reshape(_N))
